# Initial kernel scaffold; baseline (speedup 1.0000x reference)
import jax
import jax.numpy as jnp
from jax.experimental import pallas as pl


def kernel(X_sub, sub_nodes, graphsage_nbr_ids, global_to_sub, W_nbr, b_nbr, W_final, b_final):
    raise NotImplementedError("write your pallas kernel here")


# speedup vs baseline: 7.2124x; 7.2124x over previous
import functools

import jax
import jax.numpy as jnp
from jax import lax
from jax.experimental import pallas as pl
from jax.experimental.pallas import tpu as pltpu
from jax.experimental.pallas import tpu_sc as plsc

N_SUB = 10000
N_GLOBAL = 100000
K = 32
D = 128

NC = 2
NS = 16
NW = NC * NS
N_PAD = 10240
PER_W = N_PAD // NW
CH = 8
NCHUNK = PER_W // CH


def _sc_nbr_sum(X_sub, sn_pad, nbr_ids, g2s):
    mesh = plsc.VectorSubcoreMesh(core_axis_name="c", subcore_axis_name="s")

    @functools.partial(
        pl.kernel,
        mesh=mesh,
        out_type=jax.ShapeDtypeStruct((N_PAD, D), jnp.float32),
        compiler_params=pltpu.CompilerParams(use_tc_tiling_on_sc=False),
        scratch_types=[
            pltpu.VMEM((PER_W,), jnp.int32),
            pltpu.VMEM((CH, K), jnp.int32),
            pltpu.VMEM((CH, K), jnp.int32),
            pltpu.VMEM((CH * K, D), jnp.float32),
            pltpu.VMEM((CH, D), jnp.float32),
            pltpu.SemaphoreType.DMA,
            pltpu.SemaphoreType.DMA,
            pltpu.SemaphoreType.DMA,
        ],
    )
    def sc_kernel(x_hbm, sn_hbm, nbr_hbm, g2s_hbm, out_hbm,
                  snodes_v, nbrs_v, sidx_v, rows_v, ostage_v,
                  sem_a, sem_b, sem_c):
        wid = lax.axis_index("s") * NC + lax.axis_index("c")
        wbase = wid * PER_W
        pltpu.sync_copy(sn_hbm.at[pl.ds(wbase, PER_W)], snodes_v)

        @pl.loop(0, NCHUNK)
        def _(c):
            nb = c * CH
            pltpu.async_copy(
                nbr_hbm.at[snodes_v.at[pl.ds(nb, CH)]], nbrs_v, sem_a
            ).wait()
            gs = [
                pltpu.async_copy(g2s_hbm.at[nbrs_v.at[n]], sidx_v.at[n], sem_b)
                for n in range(CH)
            ]
            for cp in gs:
                cp.wait()
            xs = [
                pltpu.async_copy(
                    x_hbm.at[sidx_v.at[n]], rows_v.at[pl.ds(n * K, K)], sem_c
                )
                for n in range(CH)
            ]
            for cp in xs:
                cp.wait()

            @pl.loop(0, CH)
            def _(n):
                rbase = n * K
                for chk in range(D // 16):
                    a = rows_v[pl.ds(rbase, 1), pl.ds(chk * 16, 16)]
                    for r in range(1, K):
                        a = a + rows_v[pl.ds(rbase + r, 1), pl.ds(chk * 16, 16)]
                    ostage_v[pl.ds(n, 1), pl.ds(chk * 16, 16)] = a

            pltpu.sync_copy(ostage_v, out_hbm.at[pl.ds(wbase + nb, CH)])

    return sc_kernel(X_sub, sn_pad, nbr_ids, g2s)


def _tc_body(x_ref, s_ref, wn_ref, bn_ref, w1_ref, w2_ref, bf_ref, o_ref):
    mean = s_ref[...] * (1.0 / K)
    h = jnp.dot(mean, wn_ref[...], preferred_element_type=jnp.float32,
                precision=lax.Precision.HIGHEST)
    h = jnp.maximum(h + bn_ref[0:1, :], 0.0)
    o = (jnp.dot(x_ref[...], w1_ref[...], preferred_element_type=jnp.float32,
                 precision=lax.Precision.HIGHEST)
         + jnp.dot(h, w2_ref[...], preferred_element_type=jnp.float32,
                   precision=lax.Precision.HIGHEST)
         + bf_ref[0:1, :])
    o = jnp.maximum(o, 0.0)
    nrm = jnp.sqrt(jnp.sum(o * o, axis=1, keepdims=True))
    o_ref[...] = o / jnp.maximum(nrm, 1e-12)


def _tc_dense(X_sub, nbr_sum, Wn_t, b_nbr, W1_t, W2_t, b_final):
    blk = 2000
    grid = (N_SUB // blk,)
    return pl.pallas_call(
        _tc_body,
        grid=grid,
        in_specs=[
            pl.BlockSpec((blk, D), lambda i: (i, 0)),
            pl.BlockSpec((blk, D), lambda i: (i, 0)),
            pl.BlockSpec((D, D), lambda i: (0, 0)),
            pl.BlockSpec((8, D), lambda i: (0, 0)),
            pl.BlockSpec((D, D), lambda i: (0, 0)),
            pl.BlockSpec((D, D), lambda i: (0, 0)),
            pl.BlockSpec((8, D), lambda i: (0, 0)),
        ],
        out_specs=pl.BlockSpec((blk, D), lambda i: (i, 0)),
        out_shape=jax.ShapeDtypeStruct((N_SUB, D), jnp.float32),
    )(X_sub, nbr_sum, Wn_t, b_nbr, W1_t, W2_t, b_final)


def kernel(X_sub, sub_nodes, graphsage_nbr_ids, global_to_sub, W_nbr, b_nbr, W_final, b_final):
    sn_pad = jnp.concatenate(
        [sub_nodes, jnp.zeros((N_PAD - N_SUB,), jnp.int32)])
    nbr_sum = _sc_nbr_sum(X_sub, sn_pad, graphsage_nbr_ids, global_to_sub)
    Wn_t = W_nbr.T
    Wf_t = W_final.T
    bn = jnp.broadcast_to(b_nbr[None, :], (8, D))
    bf = jnp.broadcast_to(b_final[None, :], (8, D))
    return _tc_dense(X_sub, nbr_sum[:N_SUB], Wn_t, bn,
                     Wf_t[:D], Wf_t[D:], bf)


# speedup vs baseline: 11.3940x; 1.5798x over previous
import functools

import jax
import jax.numpy as jnp
from jax import lax
from jax.experimental import pallas as pl
from jax.experimental.pallas import tpu as pltpu
from jax.experimental.pallas import tpu_sc as plsc

N_SUB = 10000
N_GLOBAL = 100000
K = 32
D = 128

NC = 2
NS = 16
NW = NC * NS
N_PAD = 10240
PER_W = N_PAD // NW
CH = 8
NCHUNK = PER_W // CH


def _sc_nbr_sum(X_sub, sn_pad, nbr_ids, g2s):
    mesh = plsc.VectorSubcoreMesh(core_axis_name="c", subcore_axis_name="s")

    @functools.partial(
        pl.kernel,
        mesh=mesh,
        out_type=jax.ShapeDtypeStruct((N_PAD, D), jnp.float32),
        compiler_params=pltpu.CompilerParams(use_tc_tiling_on_sc=False),
        scratch_types=[
            pltpu.VMEM((PER_W,), jnp.int32),
            pltpu.VMEM((PER_W, K), jnp.int32),
            pltpu.VMEM((PER_W * K,), jnp.int32),
            pltpu.VMEM((CH * K, D), jnp.float32),
            pltpu.VMEM((CH * K, D), jnp.float32),
            pltpu.VMEM((CH, D), jnp.float32),
            pltpu.VMEM((CH, D), jnp.float32),
            pltpu.SemaphoreType.DMA,
            pltpu.SemaphoreType.DMA,
            pltpu.SemaphoreType.DMA,
            pltpu.SemaphoreType.DMA,
            pltpu.SemaphoreType.DMA,
            pltpu.SemaphoreType.DMA,
        ],
    )
    def sc_kernel(x_hbm, sn_hbm, nbr_hbm, g2s_hbm, out_hbm,
                  snodes_v, nbrs_v, sidx_v, rows0_v, rows1_v, ost0_v, ost1_v,
                  sem_a, sem_g, sem_x0, sem_x1, sem_o0, sem_o1):
        wid = lax.axis_index("s") * NC + lax.axis_index("c")
        wbase = wid * PER_W
        pltpu.sync_copy(sn_hbm.at[pl.ds(wbase, PER_W)], snodes_v)
        pltpu.async_copy(nbr_hbm.at[snodes_v], nbrs_v, sem_a).wait()

        def issue_g2s(c):
            for j in range(CH):
                pltpu.async_copy(
                    g2s_hbm.at[nbrs_v.at[c * CH + j]],
                    sidx_v.at[pl.ds((c * CH + j) * K, K)], sem_g)

        def drain_g2s():
            for _ in range(CH):
                pltpu.make_async_copy(
                    g2s_hbm.at[nbrs_v.at[0]], sidx_v.at[pl.ds(0, K)],
                    sem_g).wait()

        issue_g2s(0)
        issue_g2s(1)

        @pl.loop(0, NCHUNK)
        def _(c):
            @pl.when(c < NCHUNK - 2)
            def _():
                issue_g2s(c + 2)
            drain_g2s()

        def issue_x(c, rows_ref, sem):
            pltpu.async_copy(
                x_hbm.at[sidx_v.at[pl.ds(c * CH * K, CH * K)]], rows_ref, sem)

        def wait_x(rows_ref, sem):
            pltpu.make_async_copy(
                x_hbm.at[sidx_v.at[pl.ds(0, CH * K)]], rows_ref, sem).wait()

        def wait_out(ost_ref, sem):
            pltpu.make_async_copy(
                ost_ref, out_hbm.at[pl.ds(wbase, CH)], sem).wait()

        def reduce_chunk(rows_ref, ost_ref):
            @pl.loop(0, CH)
            def _(n):
                acc = [rows_ref[pl.ds(n * K, 1), pl.ds(chk * 16, 16)]
                       for chk in range(D // 16)]
                for r in range(1, K):
                    for chk in range(D // 16):
                        acc[chk] = acc[chk] + rows_ref[
                            pl.ds(n * K + r, 1), pl.ds(chk * 16, 16)]
                for chk in range(D // 16):
                    ost_ref[pl.ds(n, 1), pl.ds(chk * 16, 16)] = acc[chk]

        issue_x(0, rows0_v, sem_x0)
        issue_x(1, rows1_v, sem_x1)

        @pl.loop(0, NCHUNK // 2)
        def _(t):
            c0 = t * 2
            wait_x(rows0_v, sem_x0)
            @pl.when(t > 0)
            def _():
                wait_out(ost0_v, sem_o0)
            reduce_chunk(rows0_v, ost0_v)
            @pl.when(c0 + 2 < NCHUNK)
            def _():
                issue_x(c0 + 2, rows0_v, sem_x0)
            pltpu.async_copy(
                ost0_v, out_hbm.at[pl.ds(wbase + c0 * CH, CH)], sem_o0)
            wait_x(rows1_v, sem_x1)
            @pl.when(t > 0)
            def _():
                wait_out(ost1_v, sem_o1)
            reduce_chunk(rows1_v, ost1_v)
            @pl.when(c0 + 3 < NCHUNK)
            def _():
                issue_x(c0 + 3, rows1_v, sem_x1)
            pltpu.async_copy(
                ost1_v, out_hbm.at[pl.ds(wbase + (c0 + 1) * CH, CH)], sem_o1)

        wait_out(ost0_v, sem_o0)
        wait_out(ost1_v, sem_o1)

    return sc_kernel(X_sub, sn_pad, nbr_ids, g2s)


def _tc_body(x_ref, s_ref, wn_ref, bn_ref, w1_ref, w2_ref, bf_ref, o_ref):
    mean = s_ref[...] * (1.0 / K)
    h = jnp.dot(mean, wn_ref[...], preferred_element_type=jnp.float32,
                precision=lax.Precision.HIGHEST)
    h = jnp.maximum(h + bn_ref[0:1, :], 0.0)
    o = (jnp.dot(x_ref[...], w1_ref[...], preferred_element_type=jnp.float32,
                 precision=lax.Precision.HIGHEST)
         + jnp.dot(h, w2_ref[...], preferred_element_type=jnp.float32,
                   precision=lax.Precision.HIGHEST)
         + bf_ref[0:1, :])
    o = jnp.maximum(o, 0.0)
    nrm = jnp.sqrt(jnp.sum(o * o, axis=1, keepdims=True))
    o_ref[...] = o / jnp.maximum(nrm, 1e-12)


def _tc_dense(X_sub, nbr_sum, Wn_t, b_nbr, W1_t, W2_t, b_final):
    blk = 2000
    grid = (N_SUB // blk,)
    return pl.pallas_call(
        _tc_body,
        grid=grid,
        in_specs=[
            pl.BlockSpec((blk, D), lambda i: (i, 0)),
            pl.BlockSpec((blk, D), lambda i: (i, 0)),
            pl.BlockSpec((D, D), lambda i: (0, 0)),
            pl.BlockSpec((8, D), lambda i: (0, 0)),
            pl.BlockSpec((D, D), lambda i: (0, 0)),
            pl.BlockSpec((D, D), lambda i: (0, 0)),
            pl.BlockSpec((8, D), lambda i: (0, 0)),
        ],
        out_specs=pl.BlockSpec((blk, D), lambda i: (i, 0)),
        out_shape=jax.ShapeDtypeStruct((N_SUB, D), jnp.float32),
    )(X_sub, nbr_sum, Wn_t, b_nbr, W1_t, W2_t, b_final)


def kernel(X_sub, sub_nodes, graphsage_nbr_ids, global_to_sub, W_nbr, b_nbr, W_final, b_final):
    sn_pad = jnp.concatenate(
        [sub_nodes, jnp.zeros((N_PAD - N_SUB,), jnp.int32)])
    nbr_sum = _sc_nbr_sum(X_sub, sn_pad, graphsage_nbr_ids, global_to_sub)
    Wn_t = W_nbr.T
    Wf_t = W_final.T
    bn = jnp.broadcast_to(b_nbr[None, :], (8, D))
    bf = jnp.broadcast_to(b_final[None, :], (8, D))
    return _tc_dense(X_sub, nbr_sum[:N_SUB], Wn_t, bn,
                     Wf_t[:D], Wf_t[D:], bf)
